# baseline (speedup 1.0000x reference)
import functools
import numpy as np

import jax
import jax.numpy as jnp
from jax import lax
from jax.experimental import pallas as pl
from jax.experimental.pallas import tpu as pltpu
from jax.experimental.pallas import tpu_sc as plsc

N = 10000
E = 160000
D = 128
H = 2
TWO_D = 2 * D

_INV_BN = 1.0 / np.sqrt(1.0 + 1e-5)
_SCALE = 1.0 / np.sqrt(TWO_D)
_F32 = jnp.float32

_NC = 2
_NS = 16
_NW = _NC * _NS

_K = 5
_EC = E // _K
_EPW = _EC // _NW
_GC = 40
_EPT = _EC // _NS
_SC = 80
_NPAD = 10240
_NPT = _NPAD // _NS
_ZR = 128

_BN = 1000
_BE = 640


def _sigmoid(v):
    return 0.5 * jnp.tanh(0.5 * v) + 0.5


def _silu(v):
    return v * _sigmoid(v)


def _dot(a, b):
    return jnp.dot(a, b, preferred_element_type=_F32)


_BF16 = jnp.bfloat16


def _pack2(a, b):
    ua = lax.bitcast_convert_type(a.astype(_BF16), jnp.uint16).astype(jnp.uint32)
    ub = lax.bitcast_convert_type(b.astype(_BF16), jnp.uint16).astype(jnp.uint32)
    return lax.bitcast_convert_type(ua | (ub << 16), _F32)


def _unpack2(w):
    u = lax.bitcast_convert_type(w, jnp.uint32)
    lo = lax.bitcast_convert_type((u & 0xFFFF).astype(jnp.uint16), _BF16)
    hi = lax.bitcast_convert_type((u >> 16).astype(jnp.uint16), _BF16)
    return lo, hi


def _fold_body(ke_ref, ve_ref, ku1b_ref, lu1b_ref, wke_ref, wve_ref):
    wke_ref[0] = _dot(ke_ref[0], ku1b_ref[0])
    wve_ref[0] = _dot(ve_ref[0], lu1b_ref[0])


def _fold_weights(K_e2v, V_e2v, ku_w1_bot, lu_w1_bot):
    return pl.pallas_call(
        _fold_body,
        grid=(H,),
        in_specs=[
            pl.BlockSpec((1, D, D), lambda h: (h, 0, 0)),
            pl.BlockSpec((1, D, D), lambda h: (h, 0, 0)),
            pl.BlockSpec((1, D, TWO_D), lambda h: (h, 0, 0)),
            pl.BlockSpec((1, D, TWO_D), lambda h: (h, 0, 0)),
        ],
        out_specs=[
            pl.BlockSpec((1, D, TWO_D), lambda h: (h, 0, 0)),
            pl.BlockSpec((1, D, TWO_D), lambda h: (h, 0, 0)),
        ],
        out_shape=[
            jax.ShapeDtypeStruct((H, D, TWO_D), _F32),
            jax.ShapeDtypeStruct((H, D, TWO_D), _F32),
        ],
    )(K_e2v, V_e2v, ku_w1_bot, lu_w1_bot)


def _node_body(x_ref, kw_ref, vw_ref, src_ref, dst_ref):
    xb = x_ref[...].astype(_BF16)
    kvs = []
    for h in range(H):
        kv = _dot(xb, kw_ref[h])
        vv = _dot(xb, vw_ref[h])
        src_ref[:, h * D:(h + 1) * D] = _pack2(kv, vv)
        kvs.append(kv)
    dst_ref[...] = _pack2(kvs[0], kvs[1])


def _node_tables(x, K_v2v, V_v2v):
    return pl.pallas_call(
        _node_body,
        grid=(N // _BN,),
        in_specs=[
            pl.BlockSpec((_BN, D), lambda i: (i, 0)),
            pl.BlockSpec((H, D, D), lambda i: (0, 0, 0)),
            pl.BlockSpec((H, D, D), lambda i: (0, 0, 0)),
        ],
        out_specs=[
            pl.BlockSpec((_BN, TWO_D), lambda i: (i, 0)),
            pl.BlockSpec((_BN, D), lambda i: (i, 0)),
        ],
        out_shape=[
            jax.ShapeDtypeStruct((N, TWO_D), _F32),
            jax.ShapeDtypeStruct((N, D), _F32),
        ],
    )(x, K_v2v.astype(_BF16), V_v2v.astype(_BF16))


_GN = _EPW // _GC


def _gather_body(src_tab, dst_tab, src_idx, dst_idx, gsrc, gdst,
                 si, di, sbufs, dbufs, gsems, *, base0):
    wid = lax.axis_index("s") * _NC + lax.axis_index("c")
    base = wid * _EPW

    pltpu.sync_copy(src_idx.at[pl.ds(base0 + base, _EPW)], si)
    pltpu.sync_copy(dst_idx.at[pl.ds(base0 + base, _EPW)], di)

    def fire(j, b):
        idx_s = si.at[pl.ds(j * _GC, _GC)]
        idx_d = di.at[pl.ds(j * _GC, _GC)]
        pltpu.async_copy(src_tab.at[idx_s], sbufs.at[b], gsems.at[2 * b])
        pltpu.async_copy(dst_tab.at[idx_d], dbufs.at[b], gsems.at[2 * b + 1])

    def drain_write(j, b):
        idx_s = si.at[pl.ds(j * _GC, _GC)]
        idx_d = di.at[pl.ds(j * _GC, _GC)]
        pltpu.make_async_copy(src_tab.at[idx_s], sbufs.at[b],
                              gsems.at[2 * b]).wait()
        pltpu.make_async_copy(dst_tab.at[idx_d], dbufs.at[b],
                              gsems.at[2 * b + 1]).wait()
        off = base + j * _GC
        pltpu.sync_copy(sbufs.at[b], gsrc.at[pl.ds(off, _GC)])
        pltpu.sync_copy(dbufs.at[b], gdst.at[pl.ds(off, _GC)])

    fire(0, 0)

    def body(i, carry):
        p = lax.rem(i, 2)

        @pl.when(jnp.logical_and(i + 1 < _GN, p == 0))
        def _():
            fire(i + 1, 1)

        @pl.when(jnp.logical_and(i + 1 < _GN, p == 1))
        def _():
            fire(i + 1, 0)

        @pl.when(p == 0)
        def _():
            drain_write(i, 0)

        @pl.when(p == 1)
        def _():
            drain_write(i, 1)

        return carry

    lax.fori_loop(0, _GN, body, 0)


def _gather(src_tab, dst_tab, src_idx, dst_idx, k):
    mesh = plsc.VectorSubcoreMesh(core_axis_name="c", subcore_axis_name="s")
    f = functools.partial(
        pl.kernel,
        out_type=[
            jax.ShapeDtypeStruct((_EC, TWO_D), _F32),
            jax.ShapeDtypeStruct((_EC, D), _F32),
        ],
        mesh=mesh,
        scratch_types=[
            pltpu.VMEM((_EPW,), jnp.int32),
            pltpu.VMEM((_EPW,), jnp.int32),
            pltpu.VMEM((2, _GC, TWO_D), _F32),
            pltpu.VMEM((2, _GC, D), _F32),
            pltpu.SemaphoreType.DMA((4,)),
        ],
    )(functools.partial(_gather_body, base0=k * _EC))
    return f(src_tab, dst_tab, src_idx, dst_idx)


def _edge_body(ef_ref, gsrc_ref, gdst_ref,
               wk1_ref, ku2_ref, kub1_ref, kub2_ref,
               wv1_ref, lu2_ref, lub1_ref, lub2_ref,
               mlw_ref, mlb_ref, lng_ref, lnb_ref, bng_ref, bnb_ref,
               out_ref):
    ef = ef_ref[...].astype(_BF16)
    kd_lo, kd_hi = _unpack2(gdst_ref[...])
    kds = [kd_lo, kd_hi]
    for h in range(H):
        kvs, vvs = _unpack2(gsrc_ref[:, h * D:(h + 1) * D])
        bngs = (bng_ref[h] * (_SCALE * _INV_BN)).astype(_BF16)
        bnb16 = bnb_ref[h].astype(_BF16)
        gk = _dot(jnp.concatenate([kvs, ef], axis=1), wk1_ref[h])
        gk = gk.astype(_BF16) + kub1_ref[h].astype(_BF16)
        kj = _dot(_silu(gk), ku2_ref[h])
        kj = kj.astype(_BF16) + kub2_ref[h].astype(_BF16)
        gv = _dot(jnp.concatenate([vvs, ef], axis=1), wv1_ref[h])
        gv = gv.astype(_BF16) + lub1_ref[h].astype(_BF16)
        o = _dot(_silu(gv), lu2_ref[h])
        o = o.astype(_BF16) + lub2_ref[h].astype(_BF16)
        kd = kds[h]
        q2 = jnp.concatenate([kd, kd], axis=1)
        gate = _sigmoid(q2 * kj * bngs + bnb16)
        mm = _dot(o * gate, mlw_ref[h]) + mlb_ref[h]
        mu = jnp.mean(mm, axis=1, keepdims=True)
        var = jnp.mean(jnp.square(mm - mu), axis=1, keepdims=True)
        mm = (mm - mu) * lax.rsqrt(var + 1e-5) * lng_ref[h] + lnb_ref[h]
        out_ref[h] = _silu(mm)


def _edge_mlp(ef, gsrc, gdst, wk1, ku2, kub1, kub2,
              wv1, lu2, lub1, lub2, mlw, mlb, lng, lnb, bng, bnb, k):
    full3 = lambda shape: pl.BlockSpec(shape, lambda e: (0, 0, 0))
    e0 = k * (_EC // _BE)
    return pl.pallas_call(
        _edge_body,
        grid=(_EC // _BE,),
        in_specs=[
            pl.BlockSpec((_BE, D), lambda e: (e0 + e, 0)),
            pl.BlockSpec((_BE, TWO_D), lambda e: (e, 0)),
            pl.BlockSpec((_BE, D), lambda e: (e, 0)),
            full3((H, TWO_D, TWO_D)),
            full3((H, TWO_D, TWO_D)),
            full3((H, 1, TWO_D)),
            full3((H, 1, TWO_D)),
            full3((H, TWO_D, TWO_D)),
            full3((H, TWO_D, TWO_D)),
            full3((H, 1, TWO_D)),
            full3((H, 1, TWO_D)),
            full3((H, TWO_D, D)),
            full3((H, 1, D)),
            full3((H, 1, D)),
            full3((H, 1, D)),
            full3((H, 1, TWO_D)),
            full3((H, 1, TWO_D)),
        ],
        out_specs=pl.BlockSpec((H, _BE, D), lambda e: (0, e, 0)),
        out_shape=jax.ShapeDtypeStruct((H, _EC, D), _F32),
    )(ef, gsrc, gdst, wk1, ku2, kub1, kub2,
      wv1, lu2, lub1, lub2, mlw, mlb, lng, lnb, bng, bnb)


_SN = _EPT // _SC


def _scatter_group_body(*refs, k0, nm):
    ms = refs[:nm]
    dst_idx3, hv_hbm, ibuf, mbufs, zbuf, acc, msems = refs[nm:]
    cid = lax.axis_index("c")
    sid = lax.axis_index("s")
    base = sid * _EPT

    def zb_body(i, carry):
        for j in range(D // 16):
            zbuf[i, pl.ds(j * 16, 16)] = jnp.zeros((16,), _F32)
        return carry

    lax.fori_loop(0, _ZR, zb_body, 0)
    for j in range(_NPT // _ZR):
        pltpu.sync_copy(zbuf, acc.at[pl.ds(sid * _NPT + j * _ZR, _ZR)])
    plsc.subcore_barrier()

    for lk in range(nm):
        m_hbm = ms[lk]

        pltpu.sync_copy(dst_idx3.at[(k0 + lk) * _NS + sid], ibuf)

        def fire(j, b, m_hbm=m_hbm):
            pltpu.async_copy(m_hbm.at[cid, pl.ds(base + j * _SC, _SC)],
                             mbufs.at[b], msems.at[b])

        def drain_scatter(j, b, m_hbm=m_hbm):
            pltpu.make_async_copy(m_hbm.at[cid, pl.ds(base + j * _SC, _SC)],
                                  mbufs.at[b], msems.at[b]).wait()
            pltpu.sync_copy(mbufs.at[b], acc.at[ibuf.at[j]], add=True)

        fire(0, 0)

        def body(i, carry):
            p = lax.rem(i, 2)

            @pl.when(jnp.logical_and(i + 1 < _SN, p == 0))
            def _():
                fire(i + 1, 1)

            @pl.when(jnp.logical_and(i + 1 < _SN, p == 1))
            def _():
                fire(i + 1, 0)

            @pl.when(p == 0)
            def _():
                drain_scatter(i, 0)

            @pl.when(p == 1)
            def _():
                drain_scatter(i, 1)

            return carry

        lax.fori_loop(0, _SN, body, 0)

    plsc.subcore_barrier()
    pltpu.sync_copy(acc.at[pl.ds(sid * _NPT, _NPT)],
                    hv_hbm.at[cid, pl.ds(sid * _NPT, _NPT)])


def _scatter_group(ms, dst_idx, k0):
    mesh = plsc.VectorSubcoreMesh(core_axis_name="c", subcore_axis_name="s")
    f = functools.partial(
        pl.kernel,
        out_type=jax.ShapeDtypeStruct((H, _NPAD, D), _F32),
        mesh=mesh,
        scratch_types=[
            pltpu.VMEM((_SN, _SC), jnp.int32),
            pltpu.VMEM((2, _SC, D), _F32),
            pltpu.VMEM((_ZR, D), _F32),
            pltpu.VMEM_SHARED((_NPAD, D), _F32),
            pltpu.SemaphoreType.DMA((2,)),
        ],
    )(functools.partial(_scatter_group_body, k0=k0, nm=len(ms)))
    return f(*ms, dst_idx.reshape(_K * _NS, _SN, _SC))


_NHV = 2


def _final_body(*refs):
    x_ref = refs[0]
    hv_refs = refs[1:1 + _NHV]
    wc_ref, bc_ref, bnvg_ref, bnvb_ref, o_ref = refs[1 + _NHV:]
    hv0 = hv_refs[0][0]
    hv1 = hv_refs[0][1]
    for r in hv_refs[1:]:
        hv0 = hv0 + r[0]
        hv1 = hv1 + r[1]
    t = _dot(hv0, wc_ref[:D]) + _dot(hv1, wc_ref[D:]) + bc_ref[...]
    t = t * _INV_BN * bnvg_ref[...] + bnvb_ref[...]
    o_ref[...] = x_ref[...] + _silu(t)


def _final(x, hvs, wc, bc, bnvg, bnvb):
    return pl.pallas_call(
        _final_body,
        grid=(N // _BN,),
        in_specs=[pl.BlockSpec((_BN, D), lambda i: (i, 0))]
        + [pl.BlockSpec((H, _BN, D), lambda i: (0, i, 0))] * _NHV
        + [
            pl.BlockSpec((TWO_D, D), lambda i: (0, 0)),
            pl.BlockSpec((1, D), lambda i: (0, 0)),
            pl.BlockSpec((1, D), lambda i: (0, 0)),
            pl.BlockSpec((1, D), lambda i: (0, 0)),
        ],
        out_specs=pl.BlockSpec((_BN, D), lambda i: (i, 0)),
        out_shape=jax.ShapeDtypeStruct((N, D), _F32),
    )(x, *hvs, wc, bc, bnvg, bnvb)


def kernel(x, edge_index, edge_feature, K_v2v, V_v2v, K_e2v, V_e2v,
           ku_w1, ku_b1, ku_w2, ku_b2, lu_w1, lu_b1, lu_w2, lu_b2,
           ml_w, ml_b, ln_g, ln_b, bn_g, bn_b, wc, bc, bnv_g, bnv_b):
    eidx = edge_index.astype(jnp.int32)
    src_idx = eidx[0]
    dst_idx = eidx[1]

    wke, wve = _fold_weights(K_e2v, V_e2v, ku_w1[:, D:, :], lu_w1[:, D:, :])
    src_tab, dst_tab = _node_tables(x, K_v2v, V_v2v)

    r2 = lambda a: a.reshape(H, 1, a.shape[-1])
    b16 = lambda a: a.astype(_BF16)
    wk1 = jnp.concatenate([ku_w1[:, :D, :], wke], axis=1)
    wv1 = jnp.concatenate([lu_w1[:, :D, :], wve], axis=1)
    ew = (b16(wk1), b16(ku_w2), r2(ku_b1), r2(ku_b2),
          b16(wv1), b16(lu_w2), r2(lu_b1), r2(lu_b2),
          b16(ml_w), r2(ml_b), r2(ln_g), r2(ln_b), r2(bn_g), r2(bn_b))

    ms = []
    for k in range(_K):
        gsrc, gdst = _gather(src_tab, dst_tab, src_idx, dst_idx, k)
        ms.append(_edge_mlp(edge_feature, gsrc, gdst, *ew, k))
    hvs = [_scatter_group(ms[0:3], dst_idx, 0),
           _scatter_group(ms[3:5], dst_idx, 3)]

    return _final(x, hvs, wc, bc.reshape(1, D),
                  bnv_g.reshape(1, D), bnv_b.reshape(1, D))


# speedup vs baseline: 1.0485x; 1.0485x over previous
import functools
import numpy as np

import jax
import jax.numpy as jnp
from jax import lax
from jax.experimental import pallas as pl
from jax.experimental.pallas import tpu as pltpu
from jax.experimental.pallas import tpu_sc as plsc

N = 10000
E = 160000
D = 128
H = 2
TWO_D = 2 * D

_INV_BN = 1.0 / np.sqrt(1.0 + 1e-5)
_SCALE = 1.0 / np.sqrt(TWO_D)
_F32 = jnp.float32

_NC = 2
_NS = 16
_NW = _NC * _NS

_K = 5
_EC = E // _K
_EPW = _EC // _NW
_GC = 40
_EPT = _EC // _NS
_SC = 80
_NPAD = 10240
_NPT = _NPAD // _NS
_ZR = 128

_BN = 1000
_BE = 640


def _sigmoid(v):
    return 0.5 * jnp.tanh(0.5 * v) + 0.5


def _silu(v):
    return v * _sigmoid(v)


def _dot(a, b):
    return jnp.dot(a, b, preferred_element_type=_F32)


_BF16 = jnp.bfloat16


def _pack2(a, b):
    ua = lax.bitcast_convert_type(a.astype(_BF16), jnp.uint16).astype(jnp.uint32)
    ub = lax.bitcast_convert_type(b.astype(_BF16), jnp.uint16).astype(jnp.uint32)
    return lax.bitcast_convert_type(ua | (ub << 16), _F32)


def _unpack2(w):
    u = lax.bitcast_convert_type(w, jnp.uint32)
    lo = lax.bitcast_convert_type((u & 0xFFFF).astype(jnp.uint16), _BF16)
    hi = lax.bitcast_convert_type((u >> 16).astype(jnp.uint16), _BF16)
    return lo, hi


def _fold_body(ke_ref, ve_ref, ku1b_ref, lu1b_ref, wke_ref, wve_ref):
    wke_ref[0] = _dot(ke_ref[0], ku1b_ref[0])
    wve_ref[0] = _dot(ve_ref[0], lu1b_ref[0])


def _fold_weights(K_e2v, V_e2v, ku_w1_bot, lu_w1_bot):
    return pl.pallas_call(
        _fold_body,
        grid=(H,),
        in_specs=[
            pl.BlockSpec((1, D, D), lambda h: (h, 0, 0)),
            pl.BlockSpec((1, D, D), lambda h: (h, 0, 0)),
            pl.BlockSpec((1, D, TWO_D), lambda h: (h, 0, 0)),
            pl.BlockSpec((1, D, TWO_D), lambda h: (h, 0, 0)),
        ],
        out_specs=[
            pl.BlockSpec((1, D, TWO_D), lambda h: (h, 0, 0)),
            pl.BlockSpec((1, D, TWO_D), lambda h: (h, 0, 0)),
        ],
        out_shape=[
            jax.ShapeDtypeStruct((H, D, TWO_D), _F32),
            jax.ShapeDtypeStruct((H, D, TWO_D), _F32),
        ],
    )(K_e2v, V_e2v, ku_w1_bot, lu_w1_bot)


def _node_body(x_ref, kw_ref, vw_ref, src_ref, dst_ref):
    xb = x_ref[...].astype(_BF16)
    kvs = []
    for h in range(H):
        kv = _dot(xb, kw_ref[h])
        vv = _dot(xb, vw_ref[h])
        src_ref[:, h * D:(h + 1) * D] = _pack2(kv, vv)
        kvs.append(kv)
    dst_ref[...] = _pack2(kvs[0], kvs[1])


def _node_tables(x, K_v2v, V_v2v):
    return pl.pallas_call(
        _node_body,
        grid=(N // _BN,),
        in_specs=[
            pl.BlockSpec((_BN, D), lambda i: (i, 0)),
            pl.BlockSpec((H, D, D), lambda i: (0, 0, 0)),
            pl.BlockSpec((H, D, D), lambda i: (0, 0, 0)),
        ],
        out_specs=[
            pl.BlockSpec((_BN, TWO_D), lambda i: (i, 0)),
            pl.BlockSpec((_BN, D), lambda i: (i, 0)),
        ],
        out_shape=[
            jax.ShapeDtypeStruct((N, TWO_D), _F32),
            jax.ShapeDtypeStruct((N, D), _F32),
        ],
    )(x, K_v2v.astype(_BF16), V_v2v.astype(_BF16))


_GN = _EPW // _GC


def _gather_body(src_tab, dst_tab, src_idx, dst_idx, gsrc, gdst,
                 si, di, sbufs, dbufs, gsems, *, base0):
    wid = lax.axis_index("s") * _NC + lax.axis_index("c")
    base = wid * _EPW

    pltpu.sync_copy(src_idx.at[pl.ds(base0 + base, _EPW)], si)
    pltpu.sync_copy(dst_idx.at[pl.ds(base0 + base, _EPW)], di)

    def fire(j, b):
        idx_s = si.at[pl.ds(j * _GC, _GC)]
        idx_d = di.at[pl.ds(j * _GC, _GC)]
        pltpu.async_copy(src_tab.at[idx_s], sbufs.at[b], gsems.at[2 * b])
        pltpu.async_copy(dst_tab.at[idx_d], dbufs.at[b], gsems.at[2 * b + 1])

    def drain_write(j, b):
        idx_s = si.at[pl.ds(j * _GC, _GC)]
        idx_d = di.at[pl.ds(j * _GC, _GC)]
        pltpu.make_async_copy(src_tab.at[idx_s], sbufs.at[b],
                              gsems.at[2 * b]).wait()
        pltpu.make_async_copy(dst_tab.at[idx_d], dbufs.at[b],
                              gsems.at[2 * b + 1]).wait()
        off = base + j * _GC
        pltpu.sync_copy(sbufs.at[b], gsrc.at[pl.ds(off, _GC)])
        pltpu.sync_copy(dbufs.at[b], gdst.at[pl.ds(off, _GC)])

    fire(0, 0)

    def body(i, carry):
        p = lax.rem(i, 2)

        @pl.when(jnp.logical_and(i + 1 < _GN, p == 0))
        def _():
            fire(i + 1, 1)

        @pl.when(jnp.logical_and(i + 1 < _GN, p == 1))
        def _():
            fire(i + 1, 0)

        @pl.when(p == 0)
        def _():
            drain_write(i, 0)

        @pl.when(p == 1)
        def _():
            drain_write(i, 1)

        return carry

    lax.fori_loop(0, _GN, body, 0)


def _gather(src_tab, dst_tab, src_idx, dst_idx, k):
    mesh = plsc.VectorSubcoreMesh(core_axis_name="c", subcore_axis_name="s")
    f = functools.partial(
        pl.kernel,
        out_type=[
            jax.ShapeDtypeStruct((_EC, TWO_D), _F32),
            jax.ShapeDtypeStruct((_EC, D), _F32),
        ],
        mesh=mesh,
        scratch_types=[
            pltpu.VMEM((_EPW,), jnp.int32),
            pltpu.VMEM((_EPW,), jnp.int32),
            pltpu.VMEM((2, _GC, TWO_D), _F32),
            pltpu.VMEM((2, _GC, D), _F32),
            pltpu.SemaphoreType.DMA((4,)),
        ],
    )(functools.partial(_gather_body, base0=k * _EC))
    return f(src_tab, dst_tab, src_idx, dst_idx)


def _edge_body(ef_ref, gsrc_ref, gdst_ref,
               wk1_ref, ku2_ref, kub1_ref, kub2_ref,
               wv1_ref, lu2_ref, lub1_ref, lub2_ref,
               mlw_ref, mlb_ref, lng_ref, lnb_ref, bng_ref, bnb_ref,
               out_ref):
    ef = ef_ref[...].astype(_BF16)
    kd_lo, kd_hi = _unpack2(gdst_ref[...])
    kds = [kd_lo, kd_hi]
    for h in range(H):
        kvs, vvs = _unpack2(gsrc_ref[:, h * D:(h + 1) * D])
        bngs = (bng_ref[h] * (_SCALE * _INV_BN)).astype(_BF16)
        bnb16 = bnb_ref[h].astype(_BF16)
        gk = _dot(jnp.concatenate([kvs, ef], axis=1), wk1_ref[h])
        gk = gk.astype(_BF16) + kub1_ref[h].astype(_BF16)
        kj = _dot(_silu(gk), ku2_ref[h])
        kj = kj.astype(_BF16) + kub2_ref[h].astype(_BF16)
        gv = _dot(jnp.concatenate([vvs, ef], axis=1), wv1_ref[h])
        gv = gv.astype(_BF16) + lub1_ref[h].astype(_BF16)
        o = _dot(_silu(gv), lu2_ref[h])
        o = o.astype(_BF16) + lub2_ref[h].astype(_BF16)
        kd = kds[h]
        q2 = jnp.concatenate([kd, kd], axis=1)
        gate = _sigmoid(q2 * kj * bngs + bnb16)
        mm = _dot(o * gate, mlw_ref[h]) + mlb_ref[h]
        mu = jnp.mean(mm, axis=1, keepdims=True)
        var = jnp.mean(jnp.square(mm - mu), axis=1, keepdims=True)
        mm = (mm - mu) * lax.rsqrt(var + 1e-5) * lng_ref[h] + lnb_ref[h]
        out_ref[h] = _silu(mm)


def _edge_mlp(ef, gsrc, gdst, wk1, ku2, kub1, kub2,
              wv1, lu2, lub1, lub2, mlw, mlb, lng, lnb, bng, bnb, k):
    full3 = lambda shape: pl.BlockSpec(shape, lambda e: (0, 0, 0))
    e0 = k * (_EC // _BE)
    return pl.pallas_call(
        _edge_body,
        grid=(_EC // _BE,),
        in_specs=[
            pl.BlockSpec((_BE, D), lambda e: (e0 + e, 0)),
            pl.BlockSpec((_BE, TWO_D), lambda e: (e, 0)),
            pl.BlockSpec((_BE, D), lambda e: (e, 0)),
            full3((H, TWO_D, TWO_D)),
            full3((H, TWO_D, TWO_D)),
            full3((H, 1, TWO_D)),
            full3((H, 1, TWO_D)),
            full3((H, TWO_D, TWO_D)),
            full3((H, TWO_D, TWO_D)),
            full3((H, 1, TWO_D)),
            full3((H, 1, TWO_D)),
            full3((H, TWO_D, D)),
            full3((H, 1, D)),
            full3((H, 1, D)),
            full3((H, 1, D)),
            full3((H, 1, TWO_D)),
            full3((H, 1, TWO_D)),
        ],
        out_specs=pl.BlockSpec((H, _BE, D), lambda e: (0, e, 0)),
        out_shape=jax.ShapeDtypeStruct((H, _EC, D), _F32),
    )(ef, gsrc, gdst, wk1, ku2, kub1, kub2,
      wv1, lu2, lub1, lub2, mlw, mlb, lng, lnb, bng, bnb)


_SN = _EPT // _SC


def _scatter_group_body(*refs, k0, nm):
    ms = refs[:nm]
    dst_idx3, hv_hbm, ibuf, mbufs, zbuf, acc, msems = refs[nm:]
    cid = lax.axis_index("c")
    sid = lax.axis_index("s")
    base = sid * _EPT

    def zb_body(i, carry):
        for j in range(D // 16):
            zbuf[i, pl.ds(j * 16, 16)] = jnp.zeros((16,), _F32)
        return carry

    lax.fori_loop(0, _ZR, zb_body, 0)
    for j in range(_NPT // _ZR):
        pltpu.sync_copy(zbuf, acc.at[pl.ds(sid * _NPT + j * _ZR, _ZR)])
    plsc.subcore_barrier()

    for lk in range(nm):
        m_hbm = ms[lk]

        pltpu.sync_copy(dst_idx3.at[(k0 + lk) * _NS + sid], ibuf)

        def fire(j, b, m_hbm=m_hbm):
            pltpu.async_copy(m_hbm.at[cid, pl.ds(base + j * _SC, _SC)],
                             mbufs.at[b], msems.at[b])

        def drain_scatter(j, b, m_hbm=m_hbm):
            pltpu.make_async_copy(m_hbm.at[cid, pl.ds(base + j * _SC, _SC)],
                                  mbufs.at[b], msems.at[b]).wait()
            pltpu.sync_copy(mbufs.at[b], acc.at[ibuf.at[j]], add=True)

        fire(0, 0)

        def body(i, carry):
            p = lax.rem(i, 2)

            @pl.when(jnp.logical_and(i + 1 < _SN, p == 0))
            def _():
                fire(i + 1, 1)

            @pl.when(jnp.logical_and(i + 1 < _SN, p == 1))
            def _():
                fire(i + 1, 0)

            @pl.when(p == 0)
            def _():
                drain_scatter(i, 0)

            @pl.when(p == 1)
            def _():
                drain_scatter(i, 1)

            return carry

        lax.fori_loop(0, _SN, body, 0)

    plsc.subcore_barrier()
    pltpu.sync_copy(acc.at[pl.ds(sid * _NPT, _NPT)],
                    hv_hbm.at[cid, pl.ds(sid * _NPT, _NPT)])


def _scatter_group(ms, dst_idx, k0):
    mesh = plsc.VectorSubcoreMesh(core_axis_name="c", subcore_axis_name="s")
    f = functools.partial(
        pl.kernel,
        out_type=jax.ShapeDtypeStruct((H, _NPAD, D), _F32),
        mesh=mesh,
        scratch_types=[
            pltpu.VMEM((_SN, _SC), jnp.int32),
            pltpu.VMEM((2, _SC, D), _F32),
            pltpu.VMEM((_ZR, D), _F32),
            pltpu.VMEM_SHARED((_NPAD, D), _F32),
            pltpu.SemaphoreType.DMA((2,)),
        ],
    )(functools.partial(_scatter_group_body, k0=k0, nm=len(ms)))
    return f(*ms, dst_idx.reshape(_K * _NS, _SN, _SC))


_NHV = _K


def _final_body(*refs):
    x_ref = refs[0]
    hv_refs = refs[1:1 + _NHV]
    wc_ref, bc_ref, bnvg_ref, bnvb_ref, o_ref = refs[1 + _NHV:]
    hv0 = hv_refs[0][0]
    hv1 = hv_refs[0][1]
    for r in hv_refs[1:]:
        hv0 = hv0 + r[0]
        hv1 = hv1 + r[1]
    t = _dot(hv0, wc_ref[:D]) + _dot(hv1, wc_ref[D:]) + bc_ref[...]
    t = t * _INV_BN * bnvg_ref[...] + bnvb_ref[...]
    o_ref[...] = x_ref[...] + _silu(t)


def _final(x, hvs, wc, bc, bnvg, bnvb):
    return pl.pallas_call(
        _final_body,
        grid=(N // _BN,),
        in_specs=[pl.BlockSpec((_BN, D), lambda i: (i, 0))]
        + [pl.BlockSpec((H, _BN, D), lambda i: (0, i, 0))] * _NHV
        + [
            pl.BlockSpec((TWO_D, D), lambda i: (0, 0)),
            pl.BlockSpec((1, D), lambda i: (0, 0)),
            pl.BlockSpec((1, D), lambda i: (0, 0)),
            pl.BlockSpec((1, D), lambda i: (0, 0)),
        ],
        out_specs=pl.BlockSpec((_BN, D), lambda i: (i, 0)),
        out_shape=jax.ShapeDtypeStruct((N, D), _F32),
    )(x, *hvs, wc, bc, bnvg, bnvb)


def kernel(x, edge_index, edge_feature, K_v2v, V_v2v, K_e2v, V_e2v,
           ku_w1, ku_b1, ku_w2, ku_b2, lu_w1, lu_b1, lu_w2, lu_b2,
           ml_w, ml_b, ln_g, ln_b, bn_g, bn_b, wc, bc, bnv_g, bnv_b):
    eidx = edge_index.astype(jnp.int32)
    src_idx = eidx[0]
    dst_idx = eidx[1]

    wke, wve = _fold_weights(K_e2v, V_e2v, ku_w1[:, D:, :], lu_w1[:, D:, :])
    src_tab, dst_tab = _node_tables(x, K_v2v, V_v2v)

    r2 = lambda a: a.reshape(H, 1, a.shape[-1])
    b16 = lambda a: a.astype(_BF16)
    wk1 = jnp.concatenate([ku_w1[:, :D, :], wke], axis=1)
    wv1 = jnp.concatenate([lu_w1[:, :D, :], wve], axis=1)
    ew = (b16(wk1), b16(ku_w2), r2(ku_b1), r2(ku_b2),
          b16(wv1), b16(lu_w2), r2(lu_b1), r2(lu_b2),
          b16(ml_w), r2(ml_b), r2(ln_g), r2(ln_b), r2(bn_g), r2(bn_b))

    hvs = []
    for k in range(_K):
        gsrc, gdst = _gather(src_tab, dst_tab, src_idx, dst_idx, k)
        m = _edge_mlp(edge_feature, gsrc, gdst, *ew, k)
        hvs.append(_scatter_group([m], dst_idx, k))

    return _final(x, hvs, wc, bc.reshape(1, D),
                  bnv_g.reshape(1, D), bnv_b.reshape(1, D))


# speedup vs baseline: 1.2395x; 1.1822x over previous
import functools
import numpy as np

import jax
import jax.numpy as jnp
from jax import lax
from jax.experimental import pallas as pl
from jax.experimental.pallas import tpu as pltpu
from jax.experimental.pallas import tpu_sc as plsc

N = 10000
E = 160000
D = 128
H = 2
TWO_D = 2 * D

_INV_BN = 1.0 / np.sqrt(1.0 + 1e-5)
_SCALE = 1.0 / np.sqrt(TWO_D)
_F32 = jnp.float32

_NC = 2
_NS = 16
_NW = _NC * _NS

_K = 5
_EC = E // _K
_EPW = _EC // _NW
_GC = 40
_EPT = _EC // _NS
_SC = 80
_NPAD = 10240
_NPT = _NPAD // _NS
_ZR = 128

_BN = 1000
_BE = 1280


def _sigmoid(v):
    return 0.5 * jnp.tanh(0.5 * v) + 0.5


def _silu(v):
    return v * _sigmoid(v)


def _dot(a, b):
    return jnp.dot(a, b, preferred_element_type=_F32)


_BF16 = jnp.bfloat16


def _pack2(a, b):
    ua = lax.bitcast_convert_type(a.astype(_BF16), jnp.uint16).astype(jnp.uint32)
    ub = lax.bitcast_convert_type(b.astype(_BF16), jnp.uint16).astype(jnp.uint32)
    return lax.bitcast_convert_type(ua | (ub << 16), _F32)


def _unpack2(w):
    u = lax.bitcast_convert_type(w, jnp.uint32)
    lo = lax.bitcast_convert_type((u & 0xFFFF).astype(jnp.uint16), _BF16)
    hi = lax.bitcast_convert_type((u >> 16).astype(jnp.uint16), _BF16)
    return lo, hi


def _fold_body(ke_ref, ve_ref, ku1b_ref, lu1b_ref, wke_ref, wve_ref):
    wke_ref[0] = _dot(ke_ref[0], ku1b_ref[0])
    wve_ref[0] = _dot(ve_ref[0], lu1b_ref[0])


def _fold_weights(K_e2v, V_e2v, ku_w1_bot, lu_w1_bot):
    return pl.pallas_call(
        _fold_body,
        grid=(H,),
        in_specs=[
            pl.BlockSpec((1, D, D), lambda h: (h, 0, 0)),
            pl.BlockSpec((1, D, D), lambda h: (h, 0, 0)),
            pl.BlockSpec((1, D, TWO_D), lambda h: (h, 0, 0)),
            pl.BlockSpec((1, D, TWO_D), lambda h: (h, 0, 0)),
        ],
        out_specs=[
            pl.BlockSpec((1, D, TWO_D), lambda h: (h, 0, 0)),
            pl.BlockSpec((1, D, TWO_D), lambda h: (h, 0, 0)),
        ],
        out_shape=[
            jax.ShapeDtypeStruct((H, D, TWO_D), _F32),
            jax.ShapeDtypeStruct((H, D, TWO_D), _F32),
        ],
    )(K_e2v, V_e2v, ku_w1_bot, lu_w1_bot)


def _node_body(x_ref, kw_ref, vw_ref, src_ref, dst_ref):
    xb = x_ref[...].astype(_BF16)
    kvs = []
    for h in range(H):
        kv = _dot(xb, kw_ref[h])
        vv = _dot(xb, vw_ref[h])
        src_ref[:, h * D:(h + 1) * D] = _pack2(kv, vv)
        kvs.append(kv)
    dst_ref[...] = _pack2(kvs[0], kvs[1])


def _node_tables(x, K_v2v, V_v2v):
    return pl.pallas_call(
        _node_body,
        grid=(N // _BN,),
        in_specs=[
            pl.BlockSpec((_BN, D), lambda i: (i, 0)),
            pl.BlockSpec((H, D, D), lambda i: (0, 0, 0)),
            pl.BlockSpec((H, D, D), lambda i: (0, 0, 0)),
        ],
        out_specs=[
            pl.BlockSpec((_BN, TWO_D), lambda i: (i, 0)),
            pl.BlockSpec((_BN, D), lambda i: (i, 0)),
        ],
        out_shape=[
            jax.ShapeDtypeStruct((N, TWO_D), _F32),
            jax.ShapeDtypeStruct((N, D), _F32),
        ],
    )(x, K_v2v.astype(_BF16), V_v2v.astype(_BF16))


_GN = _EPW // _GC


def _gather_body(src_tab, dst_tab, src_idx, dst_idx, gsrc, gdst,
                 si, di, sbufs, dbufs, gsems, *, base0):
    wid = lax.axis_index("s") * _NC + lax.axis_index("c")
    base = wid * _EPW

    pltpu.sync_copy(src_idx.at[pl.ds(base0 + base, _EPW)], si)
    pltpu.sync_copy(dst_idx.at[pl.ds(base0 + base, _EPW)], di)

    def fire(j, b):
        idx_s = si.at[pl.ds(j * _GC, _GC)]
        idx_d = di.at[pl.ds(j * _GC, _GC)]
        pltpu.async_copy(src_tab.at[idx_s], sbufs.at[b], gsems.at[2 * b])
        pltpu.async_copy(dst_tab.at[idx_d], dbufs.at[b], gsems.at[2 * b + 1])

    def drain_write(j, b):
        idx_s = si.at[pl.ds(j * _GC, _GC)]
        idx_d = di.at[pl.ds(j * _GC, _GC)]
        pltpu.make_async_copy(src_tab.at[idx_s], sbufs.at[b],
                              gsems.at[2 * b]).wait()
        pltpu.make_async_copy(dst_tab.at[idx_d], dbufs.at[b],
                              gsems.at[2 * b + 1]).wait()
        off = base + j * _GC
        pltpu.sync_copy(sbufs.at[b], gsrc.at[pl.ds(off, _GC)])
        pltpu.sync_copy(dbufs.at[b], gdst.at[pl.ds(off, _GC)])

    fire(0, 0)

    def body(i, carry):
        p = lax.rem(i, 2)

        @pl.when(jnp.logical_and(i + 1 < _GN, p == 0))
        def _():
            fire(i + 1, 1)

        @pl.when(jnp.logical_and(i + 1 < _GN, p == 1))
        def _():
            fire(i + 1, 0)

        @pl.when(p == 0)
        def _():
            drain_write(i, 0)

        @pl.when(p == 1)
        def _():
            drain_write(i, 1)

        return carry

    lax.fori_loop(0, _GN, body, 0)


def _gather(src_tab, dst_tab, src_idx, dst_idx, k):
    mesh = plsc.VectorSubcoreMesh(core_axis_name="c", subcore_axis_name="s")
    f = functools.partial(
        pl.kernel,
        out_type=[
            jax.ShapeDtypeStruct((_EC, TWO_D), _F32),
            jax.ShapeDtypeStruct((_EC, D), _F32),
        ],
        mesh=mesh,
        scratch_types=[
            pltpu.VMEM((_EPW,), jnp.int32),
            pltpu.VMEM((_EPW,), jnp.int32),
            pltpu.VMEM((2, _GC, TWO_D), _F32),
            pltpu.VMEM((2, _GC, D), _F32),
            pltpu.SemaphoreType.DMA((4,)),
        ],
    )(functools.partial(_gather_body, base0=k * _EC))
    return f(src_tab, dst_tab, src_idx, dst_idx)


def _edge_body(ef_ref, gsrc_ref, gdst_ref,
               wk1_ref, ku2_ref, kub1_ref, kub2_ref,
               wv1_ref, lu2_ref, lub1_ref, lub2_ref,
               mlw_ref, mlb_ref, lng_ref, lnb_ref, bng_ref, bnb_ref,
               out_ref):
    ef = ef_ref[...].astype(_BF16)
    kd_lo, kd_hi = _unpack2(gdst_ref[...])
    kds = [kd_lo, kd_hi]
    for h in range(H):
        kvs, vvs = _unpack2(gsrc_ref[:, h * D:(h + 1) * D])
        bngs = (bng_ref[h] * (_SCALE * _INV_BN)).astype(_BF16)
        bnb16 = bnb_ref[h].astype(_BF16)
        gk = _dot(jnp.concatenate([kvs, ef], axis=1), wk1_ref[h])
        gk = gk.astype(_BF16) + kub1_ref[h].astype(_BF16)
        kj = _dot(_silu(gk), ku2_ref[h])
        kj = kj.astype(_BF16) + kub2_ref[h].astype(_BF16)
        gv = _dot(jnp.concatenate([vvs, ef], axis=1), wv1_ref[h])
        gv = gv.astype(_BF16) + lub1_ref[h].astype(_BF16)
        o = _dot(_silu(gv), lu2_ref[h])
        o = o.astype(_BF16) + lub2_ref[h].astype(_BF16)
        kd = kds[h]
        q2 = jnp.concatenate([kd, kd], axis=1)
        gate = _sigmoid(q2 * kj * bngs + bnb16)
        mm = _dot(o * gate, mlw_ref[h]) + mlb_ref[h]
        mu = jnp.mean(mm, axis=1, keepdims=True)
        var = jnp.mean(jnp.square(mm - mu), axis=1, keepdims=True)
        mm = (mm - mu) * lax.rsqrt(var + 1e-5) * lng_ref[h] + lnb_ref[h]
        out_ref[h] = _silu(mm)


def _edge_mlp(ef, gsrc, gdst, wk1, ku2, kub1, kub2,
              wv1, lu2, lub1, lub2, mlw, mlb, lng, lnb, bng, bnb, k):
    full3 = lambda shape: pl.BlockSpec(shape, lambda e: (0, 0, 0))
    e0 = k * (_EC // _BE)
    return pl.pallas_call(
        _edge_body,
        grid=(_EC // _BE,),
        in_specs=[
            pl.BlockSpec((_BE, D), lambda e: (e0 + e, 0)),
            pl.BlockSpec((_BE, TWO_D), lambda e: (e, 0)),
            pl.BlockSpec((_BE, D), lambda e: (e, 0)),
            full3((H, TWO_D, TWO_D)),
            full3((H, TWO_D, TWO_D)),
            full3((H, 1, TWO_D)),
            full3((H, 1, TWO_D)),
            full3((H, TWO_D, TWO_D)),
            full3((H, TWO_D, TWO_D)),
            full3((H, 1, TWO_D)),
            full3((H, 1, TWO_D)),
            full3((H, TWO_D, D)),
            full3((H, 1, D)),
            full3((H, 1, D)),
            full3((H, 1, D)),
            full3((H, 1, TWO_D)),
            full3((H, 1, TWO_D)),
        ],
        out_specs=pl.BlockSpec((H, _BE, D), lambda e: (0, e, 0)),
        out_shape=jax.ShapeDtypeStruct((H, _EC, D), _F32),
    )(ef, gsrc, gdst, wk1, ku2, kub1, kub2,
      wv1, lu2, lub1, lub2, mlw, mlb, lng, lnb, bng, bnb)


_SN = _EPT // _SC


def _scatter_group_body(*refs, k0, nm):
    ms = refs[:nm]
    dst_idx3, hv_hbm, ibuf, mbufs, zbuf, acc, msems = refs[nm:]
    cid = lax.axis_index("c")
    sid = lax.axis_index("s")
    base = sid * _EPT

    def zb_body(i, carry):
        for j in range(D // 16):
            zbuf[i, pl.ds(j * 16, 16)] = jnp.zeros((16,), _F32)
        return carry

    lax.fori_loop(0, _ZR, zb_body, 0)
    for j in range(_NPT // _ZR):
        pltpu.sync_copy(zbuf, acc.at[pl.ds(sid * _NPT + j * _ZR, _ZR)])
    plsc.subcore_barrier()

    for lk in range(nm):
        m_hbm = ms[lk]

        pltpu.sync_copy(dst_idx3.at[(k0 + lk) * _NS + sid], ibuf)

        def fire(j, b, m_hbm=m_hbm):
            pltpu.async_copy(m_hbm.at[cid, pl.ds(base + j * _SC, _SC)],
                             mbufs.at[b], msems.at[b])

        def drain_scatter(j, b, m_hbm=m_hbm):
            pltpu.make_async_copy(m_hbm.at[cid, pl.ds(base + j * _SC, _SC)],
                                  mbufs.at[b], msems.at[b]).wait()
            pltpu.sync_copy(mbufs.at[b], acc.at[ibuf.at[j]], add=True)

        fire(0, 0)

        def body(i, carry):
            p = lax.rem(i, 2)

            @pl.when(jnp.logical_and(i + 1 < _SN, p == 0))
            def _():
                fire(i + 1, 1)

            @pl.when(jnp.logical_and(i + 1 < _SN, p == 1))
            def _():
                fire(i + 1, 0)

            @pl.when(p == 0)
            def _():
                drain_scatter(i, 0)

            @pl.when(p == 1)
            def _():
                drain_scatter(i, 1)

            return carry

        lax.fori_loop(0, _SN, body, 0)

    plsc.subcore_barrier()
    pltpu.sync_copy(acc.at[pl.ds(sid * _NPT, _NPT)],
                    hv_hbm.at[cid, pl.ds(sid * _NPT, _NPT)])


def _scatter_group(ms, dst_idx, k0):
    mesh = plsc.VectorSubcoreMesh(core_axis_name="c", subcore_axis_name="s")
    f = functools.partial(
        pl.kernel,
        out_type=jax.ShapeDtypeStruct((H, _NPAD, D), _F32),
        mesh=mesh,
        scratch_types=[
            pltpu.VMEM((_SN, _SC), jnp.int32),
            pltpu.VMEM((2, _SC, D), _F32),
            pltpu.VMEM((_ZR, D), _F32),
            pltpu.VMEM_SHARED((_NPAD, D), _F32),
            pltpu.SemaphoreType.DMA((2,)),
        ],
    )(functools.partial(_scatter_group_body, k0=k0, nm=len(ms)))
    return f(*ms, dst_idx.reshape(_K * _NS, _SN, _SC))


_NHV = _K


def _final_body(*refs):
    x_ref = refs[0]
    hv_refs = refs[1:1 + _NHV]
    wc_ref, bc_ref, bnvg_ref, bnvb_ref, o_ref = refs[1 + _NHV:]
    hv0 = hv_refs[0][0]
    hv1 = hv_refs[0][1]
    for r in hv_refs[1:]:
        hv0 = hv0 + r[0]
        hv1 = hv1 + r[1]
    t = _dot(hv0, wc_ref[:D]) + _dot(hv1, wc_ref[D:]) + bc_ref[...]
    t = t * _INV_BN * bnvg_ref[...] + bnvb_ref[...]
    o_ref[...] = x_ref[...] + _silu(t)


def _final(x, hvs, wc, bc, bnvg, bnvb):
    return pl.pallas_call(
        _final_body,
        grid=(N // _BN,),
        in_specs=[pl.BlockSpec((_BN, D), lambda i: (i, 0))]
        + [pl.BlockSpec((H, _BN, D), lambda i: (0, i, 0))] * _NHV
        + [
            pl.BlockSpec((TWO_D, D), lambda i: (0, 0)),
            pl.BlockSpec((1, D), lambda i: (0, 0)),
            pl.BlockSpec((1, D), lambda i: (0, 0)),
            pl.BlockSpec((1, D), lambda i: (0, 0)),
        ],
        out_specs=pl.BlockSpec((_BN, D), lambda i: (i, 0)),
        out_shape=jax.ShapeDtypeStruct((N, D), _F32),
    )(x, *hvs, wc, bc, bnvg, bnvb)


def kernel(x, edge_index, edge_feature, K_v2v, V_v2v, K_e2v, V_e2v,
           ku_w1, ku_b1, ku_w2, ku_b2, lu_w1, lu_b1, lu_w2, lu_b2,
           ml_w, ml_b, ln_g, ln_b, bn_g, bn_b, wc, bc, bnv_g, bnv_b):
    eidx = edge_index.astype(jnp.int32)
    src_idx = eidx[0]
    dst_idx = eidx[1]

    wke, wve = _fold_weights(K_e2v, V_e2v, ku_w1[:, D:, :], lu_w1[:, D:, :])
    src_tab, dst_tab = _node_tables(x, K_v2v, V_v2v)

    r2 = lambda a: a.reshape(H, 1, a.shape[-1])
    b16 = lambda a: a.astype(_BF16)
    wk1 = jnp.concatenate([ku_w1[:, :D, :], wke], axis=1)
    wv1 = jnp.concatenate([lu_w1[:, :D, :], wve], axis=1)
    ew = (b16(wk1), b16(ku_w2), r2(ku_b1), r2(ku_b2),
          b16(wv1), b16(lu_w2), r2(lu_b1), r2(lu_b2),
          b16(ml_w), r2(ml_b), r2(ln_g), r2(ln_b), r2(bn_g), r2(bn_b))

    hvs = []
    for k in range(_K):
        gsrc, gdst = _gather(src_tab, dst_tab, src_idx, dst_idx, k)
        m = _edge_mlp(edge_feature, gsrc, gdst, *ew, k)
        hvs.append(_scatter_group([m], dst_idx, k))

    return _final(x, hvs, wc, bc.reshape(1, D),
                  bnv_g.reshape(1, D), bnv_b.reshape(1, D))


# speedup vs baseline: 1.2514x; 1.0096x over previous
import functools
import numpy as np

import jax
import jax.numpy as jnp
from jax import lax
from jax.experimental import pallas as pl
from jax.experimental.pallas import tpu as pltpu
from jax.experimental.pallas import tpu_sc as plsc

N = 10000
E = 160000
D = 128
H = 2
TWO_D = 2 * D

_INV_BN = 1.0 / np.sqrt(1.0 + 1e-5)
_SCALE = 1.0 / np.sqrt(TWO_D)
_F32 = jnp.float32

_NC = 2
_NS = 16
_NW = _NC * _NS

_K = 5
_EC = E // _K
_EPW = _EC // _NW
_GC = 40
_EPT = _EC // _NS
_SC = 80
_NPAD = 10240
_NPT = _NPAD // _NS
_ZR = 128

_BN = 1000
_BE = 1600


def _sigmoid(v):
    return 0.5 * jnp.tanh(0.5 * v) + 0.5


def _silu(v):
    return v * _sigmoid(v)


def _dot(a, b):
    return jnp.dot(a, b, preferred_element_type=_F32)


_BF16 = jnp.bfloat16


def _pack2(a, b):
    ua = lax.bitcast_convert_type(a.astype(_BF16), jnp.uint16).astype(jnp.uint32)
    ub = lax.bitcast_convert_type(b.astype(_BF16), jnp.uint16).astype(jnp.uint32)
    return lax.bitcast_convert_type(ua | (ub << 16), _F32)


def _unpack2(w):
    u = lax.bitcast_convert_type(w, jnp.uint32)
    lo = lax.bitcast_convert_type((u & 0xFFFF).astype(jnp.uint16), _BF16)
    hi = lax.bitcast_convert_type((u >> 16).astype(jnp.uint16), _BF16)
    return lo, hi


def _fold_body(ke_ref, ve_ref, ku1b_ref, lu1b_ref, wke_ref, wve_ref):
    wke_ref[0] = _dot(ke_ref[0], ku1b_ref[0])
    wve_ref[0] = _dot(ve_ref[0], lu1b_ref[0])


def _fold_weights(K_e2v, V_e2v, ku_w1_bot, lu_w1_bot):
    return pl.pallas_call(
        _fold_body,
        grid=(H,),
        in_specs=[
            pl.BlockSpec((1, D, D), lambda h: (h, 0, 0)),
            pl.BlockSpec((1, D, D), lambda h: (h, 0, 0)),
            pl.BlockSpec((1, D, TWO_D), lambda h: (h, 0, 0)),
            pl.BlockSpec((1, D, TWO_D), lambda h: (h, 0, 0)),
        ],
        out_specs=[
            pl.BlockSpec((1, D, TWO_D), lambda h: (h, 0, 0)),
            pl.BlockSpec((1, D, TWO_D), lambda h: (h, 0, 0)),
        ],
        out_shape=[
            jax.ShapeDtypeStruct((H, D, TWO_D), _F32),
            jax.ShapeDtypeStruct((H, D, TWO_D), _F32),
        ],
    )(K_e2v, V_e2v, ku_w1_bot, lu_w1_bot)


def _node_body(x_ref, kw_ref, vw_ref, src_ref, dst_ref):
    xb = x_ref[...].astype(_BF16)
    kvs = []
    for h in range(H):
        kv = _dot(xb, kw_ref[h])
        vv = _dot(xb, vw_ref[h])
        src_ref[:, h * D:(h + 1) * D] = _pack2(kv, vv)
        kvs.append(kv)
    dst_ref[...] = _pack2(kvs[0], kvs[1])


def _node_tables(x, K_v2v, V_v2v):
    return pl.pallas_call(
        _node_body,
        grid=(N // _BN,),
        in_specs=[
            pl.BlockSpec((_BN, D), lambda i: (i, 0)),
            pl.BlockSpec((H, D, D), lambda i: (0, 0, 0)),
            pl.BlockSpec((H, D, D), lambda i: (0, 0, 0)),
        ],
        out_specs=[
            pl.BlockSpec((_BN, TWO_D), lambda i: (i, 0)),
            pl.BlockSpec((_BN, D), lambda i: (i, 0)),
        ],
        out_shape=[
            jax.ShapeDtypeStruct((N, TWO_D), _F32),
            jax.ShapeDtypeStruct((N, D), _F32),
        ],
    )(x, K_v2v.astype(_BF16), V_v2v.astype(_BF16))


_GN = _EPW // _GC


def _gather_body(src_tab, dst_tab, src_idx, dst_idx, gsrc, gdst,
                 si, di, sbufs, dbufs, gsems, *, base0):
    wid = lax.axis_index("s") * _NC + lax.axis_index("c")
    base = wid * _EPW

    pltpu.sync_copy(src_idx.at[pl.ds(base0 + base, _EPW)], si)
    pltpu.sync_copy(dst_idx.at[pl.ds(base0 + base, _EPW)], di)

    def fire(j, b):
        idx_s = si.at[pl.ds(j * _GC, _GC)]
        idx_d = di.at[pl.ds(j * _GC, _GC)]
        pltpu.async_copy(src_tab.at[idx_s], sbufs.at[b], gsems.at[2 * b])
        pltpu.async_copy(dst_tab.at[idx_d], dbufs.at[b], gsems.at[2 * b + 1])

    def drain_write(j, b):
        idx_s = si.at[pl.ds(j * _GC, _GC)]
        idx_d = di.at[pl.ds(j * _GC, _GC)]
        pltpu.make_async_copy(src_tab.at[idx_s], sbufs.at[b],
                              gsems.at[2 * b]).wait()
        pltpu.make_async_copy(dst_tab.at[idx_d], dbufs.at[b],
                              gsems.at[2 * b + 1]).wait()
        off = base + j * _GC
        pltpu.sync_copy(sbufs.at[b], gsrc.at[pl.ds(off, _GC)])
        pltpu.sync_copy(dbufs.at[b], gdst.at[pl.ds(off, _GC)])

    fire(0, 0)

    def body(i, carry):
        p = lax.rem(i, 2)

        @pl.when(jnp.logical_and(i + 1 < _GN, p == 0))
        def _():
            fire(i + 1, 1)

        @pl.when(jnp.logical_and(i + 1 < _GN, p == 1))
        def _():
            fire(i + 1, 0)

        @pl.when(p == 0)
        def _():
            drain_write(i, 0)

        @pl.when(p == 1)
        def _():
            drain_write(i, 1)

        return carry

    lax.fori_loop(0, _GN, body, 0)


def _gather(src_tab, dst_tab, src_idx, dst_idx, k):
    mesh = plsc.VectorSubcoreMesh(core_axis_name="c", subcore_axis_name="s")
    f = functools.partial(
        pl.kernel,
        out_type=[
            jax.ShapeDtypeStruct((_EC, TWO_D), _F32),
            jax.ShapeDtypeStruct((_EC, D), _F32),
        ],
        mesh=mesh,
        scratch_types=[
            pltpu.VMEM((_EPW,), jnp.int32),
            pltpu.VMEM((_EPW,), jnp.int32),
            pltpu.VMEM((2, _GC, TWO_D), _F32),
            pltpu.VMEM((2, _GC, D), _F32),
            pltpu.SemaphoreType.DMA((4,)),
        ],
    )(functools.partial(_gather_body, base0=k * _EC))
    return f(src_tab, dst_tab, src_idx, dst_idx)


def _edge_body(ef_ref, gsrc_ref, gdst_ref,
               wk1_ref, ku2_ref, kub1_ref, kub2_ref,
               wv1_ref, lu2_ref, lub1_ref, lub2_ref,
               mlw_ref, mlb_ref, lng_ref, lnb_ref, bng_ref, bnb_ref,
               out_ref):
    ef = ef_ref[...].astype(_BF16)
    kd_lo, kd_hi = _unpack2(gdst_ref[...])
    kds = [kd_lo, kd_hi]
    for h in range(H):
        kvs, vvs = _unpack2(gsrc_ref[:, h * D:(h + 1) * D])
        bngs = (bng_ref[h] * (_SCALE * _INV_BN)).astype(_BF16)
        bnb16 = bnb_ref[h].astype(_BF16)
        gk = _dot(jnp.concatenate([kvs, ef], axis=1), wk1_ref[h])
        gk = gk.astype(_BF16) + kub1_ref[h].astype(_BF16)
        kj = _dot(_silu(gk), ku2_ref[h])
        kj = kj.astype(_BF16) + kub2_ref[h].astype(_BF16)
        gv = _dot(jnp.concatenate([vvs, ef], axis=1), wv1_ref[h])
        gv = gv.astype(_BF16) + lub1_ref[h].astype(_BF16)
        o = _dot(_silu(gv), lu2_ref[h])
        o = o.astype(_BF16) + lub2_ref[h].astype(_BF16)
        kd = kds[h]
        q2 = jnp.concatenate([kd, kd], axis=1)
        gate = _sigmoid(q2 * kj * bngs + bnb16)
        mm = _dot(o * gate, mlw_ref[h]) + mlb_ref[h]
        mu = jnp.mean(mm, axis=1, keepdims=True)
        var = jnp.mean(jnp.square(mm - mu), axis=1, keepdims=True)
        mm = (mm - mu) * lax.rsqrt(var + 1e-5) * lng_ref[h] + lnb_ref[h]
        out_ref[h] = _silu(mm)


def _edge_mlp(ef, gsrc, gdst, wk1, ku2, kub1, kub2,
              wv1, lu2, lub1, lub2, mlw, mlb, lng, lnb, bng, bnb, k):
    full3 = lambda shape: pl.BlockSpec(shape, lambda e: (0, 0, 0))
    e0 = k * (_EC // _BE)
    return pl.pallas_call(
        _edge_body,
        grid=(_EC // _BE,),
        in_specs=[
            pl.BlockSpec((_BE, D), lambda e: (e0 + e, 0)),
            pl.BlockSpec((_BE, TWO_D), lambda e: (e, 0)),
            pl.BlockSpec((_BE, D), lambda e: (e, 0)),
            full3((H, TWO_D, TWO_D)),
            full3((H, TWO_D, TWO_D)),
            full3((H, 1, TWO_D)),
            full3((H, 1, TWO_D)),
            full3((H, TWO_D, TWO_D)),
            full3((H, TWO_D, TWO_D)),
            full3((H, 1, TWO_D)),
            full3((H, 1, TWO_D)),
            full3((H, TWO_D, D)),
            full3((H, 1, D)),
            full3((H, 1, D)),
            full3((H, 1, D)),
            full3((H, 1, TWO_D)),
            full3((H, 1, TWO_D)),
        ],
        out_specs=pl.BlockSpec((H, _BE, D), lambda e: (0, e, 0)),
        out_shape=jax.ShapeDtypeStruct((H, _EC, D), _F32),
    )(ef, gsrc, gdst, wk1, ku2, kub1, kub2,
      wv1, lu2, lub1, lub2, mlw, mlb, lng, lnb, bng, bnb)


_SN = _EPT // _SC


def _scatter_group_body(*refs, k0, nm):
    ms = refs[:nm]
    dst_idx3, hv_hbm, ibuf, mbufs, zbuf, acc, msems = refs[nm:]
    cid = lax.axis_index("c")
    sid = lax.axis_index("s")
    base = sid * _EPT

    def zb_body(i, carry):
        for j in range(D // 16):
            zbuf[i, pl.ds(j * 16, 16)] = jnp.zeros((16,), _F32)
        return carry

    lax.fori_loop(0, _ZR, zb_body, 0)
    for j in range(_NPT // _ZR):
        pltpu.sync_copy(zbuf, acc.at[pl.ds(sid * _NPT + j * _ZR, _ZR)])
    plsc.subcore_barrier()

    for lk in range(nm):
        m_hbm = ms[lk]

        pltpu.sync_copy(dst_idx3.at[(k0 + lk) * _NS + sid], ibuf)

        def fire(j, b, m_hbm=m_hbm):
            pltpu.async_copy(m_hbm.at[cid, pl.ds(base + j * _SC, _SC)],
                             mbufs.at[b], msems.at[b])

        def drain_scatter(j, b, m_hbm=m_hbm):
            pltpu.make_async_copy(m_hbm.at[cid, pl.ds(base + j * _SC, _SC)],
                                  mbufs.at[b], msems.at[b]).wait()
            pltpu.sync_copy(mbufs.at[b], acc.at[ibuf.at[j]], add=True)

        fire(0, 0)

        def body(i, carry):
            p = lax.rem(i, 2)

            @pl.when(jnp.logical_and(i + 1 < _SN, p == 0))
            def _():
                fire(i + 1, 1)

            @pl.when(jnp.logical_and(i + 1 < _SN, p == 1))
            def _():
                fire(i + 1, 0)

            @pl.when(p == 0)
            def _():
                drain_scatter(i, 0)

            @pl.when(p == 1)
            def _():
                drain_scatter(i, 1)

            return carry

        lax.fori_loop(0, _SN, body, 0)

    plsc.subcore_barrier()
    pltpu.sync_copy(acc.at[pl.ds(sid * _NPT, _NPT)],
                    hv_hbm.at[cid, pl.ds(sid * _NPT, _NPT)])


def _scatter_group(ms, dst_idx, k0):
    mesh = plsc.VectorSubcoreMesh(core_axis_name="c", subcore_axis_name="s")
    f = functools.partial(
        pl.kernel,
        out_type=jax.ShapeDtypeStruct((H, _NPAD, D), _F32),
        mesh=mesh,
        scratch_types=[
            pltpu.VMEM((_SN, _SC), jnp.int32),
            pltpu.VMEM((2, _SC, D), _F32),
            pltpu.VMEM((_ZR, D), _F32),
            pltpu.VMEM_SHARED((_NPAD, D), _F32),
            pltpu.SemaphoreType.DMA((2,)),
        ],
    )(functools.partial(_scatter_group_body, k0=k0, nm=len(ms)))
    return f(*ms, dst_idx.reshape(_K * _NS, _SN, _SC))


_NHV = _K


def _final_body(*refs):
    x_ref = refs[0]
    hv_refs = refs[1:1 + _NHV]
    wc_ref, bc_ref, bnvg_ref, bnvb_ref, o_ref = refs[1 + _NHV:]
    hv0 = hv_refs[0][0]
    hv1 = hv_refs[0][1]
    for r in hv_refs[1:]:
        hv0 = hv0 + r[0]
        hv1 = hv1 + r[1]
    t = _dot(hv0, wc_ref[:D]) + _dot(hv1, wc_ref[D:]) + bc_ref[...]
    t = t * _INV_BN * bnvg_ref[...] + bnvb_ref[...]
    o_ref[...] = x_ref[...] + _silu(t)


def _final(x, hvs, wc, bc, bnvg, bnvb):
    return pl.pallas_call(
        _final_body,
        grid=(N // _BN,),
        in_specs=[pl.BlockSpec((_BN, D), lambda i: (i, 0))]
        + [pl.BlockSpec((H, _BN, D), lambda i: (0, i, 0))] * _NHV
        + [
            pl.BlockSpec((TWO_D, D), lambda i: (0, 0)),
            pl.BlockSpec((1, D), lambda i: (0, 0)),
            pl.BlockSpec((1, D), lambda i: (0, 0)),
            pl.BlockSpec((1, D), lambda i: (0, 0)),
        ],
        out_specs=pl.BlockSpec((_BN, D), lambda i: (i, 0)),
        out_shape=jax.ShapeDtypeStruct((N, D), _F32),
    )(x, *hvs, wc, bc, bnvg, bnvb)


def kernel(x, edge_index, edge_feature, K_v2v, V_v2v, K_e2v, V_e2v,
           ku_w1, ku_b1, ku_w2, ku_b2, lu_w1, lu_b1, lu_w2, lu_b2,
           ml_w, ml_b, ln_g, ln_b, bn_g, bn_b, wc, bc, bnv_g, bnv_b):
    eidx = edge_index.astype(jnp.int32)
    src_idx = eidx[0]
    dst_idx = eidx[1]

    wke, wve = _fold_weights(K_e2v, V_e2v, ku_w1[:, D:, :], lu_w1[:, D:, :])
    src_tab, dst_tab = _node_tables(x, K_v2v, V_v2v)

    r2 = lambda a: a.reshape(H, 1, a.shape[-1])
    b16 = lambda a: a.astype(_BF16)
    wk1 = jnp.concatenate([ku_w1[:, :D, :], wke], axis=1)
    wv1 = jnp.concatenate([lu_w1[:, :D, :], wve], axis=1)
    ew = (b16(wk1), b16(ku_w2), r2(ku_b1), r2(ku_b2),
          b16(wv1), b16(lu_w2), r2(lu_b1), r2(lu_b2),
          b16(ml_w), r2(ml_b), r2(ln_g), r2(ln_b), r2(bn_g), r2(bn_b))

    hvs = []
    for k in range(_K):
        gsrc, gdst = _gather(src_tab, dst_tab, src_idx, dst_idx, k)
        m = _edge_mlp(edge_feature, gsrc, gdst, *ew, k)
        hvs.append(_scatter_group([m], dst_idx, k))

    return _final(x, hvs, wc, bc.reshape(1, D),
                  bnv_g.reshape(1, D), bnv_b.reshape(1, D))


# speedup vs baseline: 1.2606x; 1.0073x over previous
import functools
import numpy as np

import jax
import jax.numpy as jnp
from jax import lax
from jax.experimental import pallas as pl
from jax.experimental.pallas import tpu as pltpu
from jax.experimental.pallas import tpu_sc as plsc

N = 10000
E = 160000
D = 128
H = 2
TWO_D = 2 * D

_INV_BN = 1.0 / np.sqrt(1.0 + 1e-5)
_SCALE = 1.0 / np.sqrt(TWO_D)
_F32 = jnp.float32

_NC = 2
_NS = 16
_NW = _NC * _NS

_K = 5
_EC = E // _K
_EPW = _EC // _NW
_GC = 40
_EPT = _EC // _NS
_SC = 80
_NPAD = 10240
_NPT = _NPAD // _NS
_ZR = 128

_BN = 1000
_BE = 3200


def _sigmoid(v):
    return 0.5 * jnp.tanh(0.5 * v) + 0.5


def _silu(v):
    return v * _sigmoid(v)


def _dot(a, b):
    return jnp.dot(a, b, preferred_element_type=_F32)


_BF16 = jnp.bfloat16


def _pack2(a, b):
    ua = lax.bitcast_convert_type(a.astype(_BF16), jnp.uint16).astype(jnp.uint32)
    ub = lax.bitcast_convert_type(b.astype(_BF16), jnp.uint16).astype(jnp.uint32)
    return lax.bitcast_convert_type(ua | (ub << 16), _F32)


def _unpack2(w):
    u = lax.bitcast_convert_type(w, jnp.uint32)
    lo = lax.bitcast_convert_type((u & 0xFFFF).astype(jnp.uint16), _BF16)
    hi = lax.bitcast_convert_type((u >> 16).astype(jnp.uint16), _BF16)
    return lo, hi


def _fold_body(ke_ref, ve_ref, ku1b_ref, lu1b_ref, wke_ref, wve_ref):
    wke_ref[0] = _dot(ke_ref[0], ku1b_ref[0])
    wve_ref[0] = _dot(ve_ref[0], lu1b_ref[0])


def _fold_weights(K_e2v, V_e2v, ku_w1_bot, lu_w1_bot):
    return pl.pallas_call(
        _fold_body,
        grid=(H,),
        in_specs=[
            pl.BlockSpec((1, D, D), lambda h: (h, 0, 0)),
            pl.BlockSpec((1, D, D), lambda h: (h, 0, 0)),
            pl.BlockSpec((1, D, TWO_D), lambda h: (h, 0, 0)),
            pl.BlockSpec((1, D, TWO_D), lambda h: (h, 0, 0)),
        ],
        out_specs=[
            pl.BlockSpec((1, D, TWO_D), lambda h: (h, 0, 0)),
            pl.BlockSpec((1, D, TWO_D), lambda h: (h, 0, 0)),
        ],
        out_shape=[
            jax.ShapeDtypeStruct((H, D, TWO_D), _F32),
            jax.ShapeDtypeStruct((H, D, TWO_D), _F32),
        ],
    )(K_e2v, V_e2v, ku_w1_bot, lu_w1_bot)


def _node_body(x_ref, kw_ref, vw_ref, src_ref, dst_ref):
    xb = x_ref[...].astype(_BF16)
    kvs = []
    for h in range(H):
        kv = _dot(xb, kw_ref[h])
        vv = _dot(xb, vw_ref[h])
        src_ref[:, h * D:(h + 1) * D] = _pack2(kv, vv)
        kvs.append(kv)
    dst_ref[...] = _pack2(kvs[0], kvs[1])


def _node_tables(x, K_v2v, V_v2v):
    return pl.pallas_call(
        _node_body,
        grid=(N // _BN,),
        in_specs=[
            pl.BlockSpec((_BN, D), lambda i: (i, 0)),
            pl.BlockSpec((H, D, D), lambda i: (0, 0, 0)),
            pl.BlockSpec((H, D, D), lambda i: (0, 0, 0)),
        ],
        out_specs=[
            pl.BlockSpec((_BN, TWO_D), lambda i: (i, 0)),
            pl.BlockSpec((_BN, D), lambda i: (i, 0)),
        ],
        out_shape=[
            jax.ShapeDtypeStruct((N, TWO_D), _F32),
            jax.ShapeDtypeStruct((N, D), _F32),
        ],
    )(x, K_v2v.astype(_BF16), V_v2v.astype(_BF16))


_GN = _EPW // _GC


def _gather_body(src_tab, dst_tab, src_idx, dst_idx, gsrc, gdst,
                 si, di, sbufs, dbufs, gsems, *, base0):
    wid = lax.axis_index("s") * _NC + lax.axis_index("c")
    base = wid * _EPW

    pltpu.sync_copy(src_idx.at[pl.ds(base0 + base, _EPW)], si)
    pltpu.sync_copy(dst_idx.at[pl.ds(base0 + base, _EPW)], di)

    def fire(j, b):
        idx_s = si.at[pl.ds(j * _GC, _GC)]
        idx_d = di.at[pl.ds(j * _GC, _GC)]
        pltpu.async_copy(src_tab.at[idx_s], sbufs.at[b], gsems.at[2 * b])
        pltpu.async_copy(dst_tab.at[idx_d], dbufs.at[b], gsems.at[2 * b + 1])

    def drain_write(j, b):
        idx_s = si.at[pl.ds(j * _GC, _GC)]
        idx_d = di.at[pl.ds(j * _GC, _GC)]
        pltpu.make_async_copy(src_tab.at[idx_s], sbufs.at[b],
                              gsems.at[2 * b]).wait()
        pltpu.make_async_copy(dst_tab.at[idx_d], dbufs.at[b],
                              gsems.at[2 * b + 1]).wait()
        off = base + j * _GC
        pltpu.sync_copy(sbufs.at[b], gsrc.at[pl.ds(off, _GC)])
        pltpu.sync_copy(dbufs.at[b], gdst.at[pl.ds(off, _GC)])

    fire(0, 0)

    def body(i, carry):
        p = lax.rem(i, 2)

        @pl.when(jnp.logical_and(i + 1 < _GN, p == 0))
        def _():
            fire(i + 1, 1)

        @pl.when(jnp.logical_and(i + 1 < _GN, p == 1))
        def _():
            fire(i + 1, 0)

        @pl.when(p == 0)
        def _():
            drain_write(i, 0)

        @pl.when(p == 1)
        def _():
            drain_write(i, 1)

        return carry

    lax.fori_loop(0, _GN, body, 0)


def _gather(src_tab, dst_tab, src_idx, dst_idx, k):
    mesh = plsc.VectorSubcoreMesh(core_axis_name="c", subcore_axis_name="s")
    f = functools.partial(
        pl.kernel,
        out_type=[
            jax.ShapeDtypeStruct((_EC, TWO_D), _F32),
            jax.ShapeDtypeStruct((_EC, D), _F32),
        ],
        mesh=mesh,
        scratch_types=[
            pltpu.VMEM((_EPW,), jnp.int32),
            pltpu.VMEM((_EPW,), jnp.int32),
            pltpu.VMEM((2, _GC, TWO_D), _F32),
            pltpu.VMEM((2, _GC, D), _F32),
            pltpu.SemaphoreType.DMA((4,)),
        ],
    )(functools.partial(_gather_body, base0=k * _EC))
    return f(src_tab, dst_tab, src_idx, dst_idx)


def _edge_body(ef_ref, gsrc_ref, gdst_ref,
               wk1_ref, ku2_ref, kub1_ref, kub2_ref,
               wv1_ref, lu2_ref, lub1_ref, lub2_ref,
               mlw_ref, mlb_ref, lng_ref, lnb_ref, bng_ref, bnb_ref,
               out_ref):
    ef = ef_ref[...].astype(_BF16)
    kd_lo, kd_hi = _unpack2(gdst_ref[...])
    kds = [kd_lo, kd_hi]
    for h in range(H):
        kvs, vvs = _unpack2(gsrc_ref[:, h * D:(h + 1) * D])
        bngs = (bng_ref[h] * (_SCALE * _INV_BN)).astype(_BF16)
        bnb16 = bnb_ref[h].astype(_BF16)
        gk = _dot(jnp.concatenate([kvs, ef], axis=1), wk1_ref[h])
        gk = gk.astype(_BF16) + kub1_ref[h].astype(_BF16)
        kj = _dot(_silu(gk), ku2_ref[h])
        kj = kj.astype(_BF16) + kub2_ref[h].astype(_BF16)
        gv = _dot(jnp.concatenate([vvs, ef], axis=1), wv1_ref[h])
        gv = gv.astype(_BF16) + lub1_ref[h].astype(_BF16)
        o = _dot(_silu(gv), lu2_ref[h])
        o = o.astype(_BF16) + lub2_ref[h].astype(_BF16)
        kd = kds[h]
        q2 = jnp.concatenate([kd, kd], axis=1)
        gate = _sigmoid(q2 * kj * bngs + bnb16)
        mm = _dot(o * gate, mlw_ref[h]) + mlb_ref[h]
        mu = jnp.mean(mm, axis=1, keepdims=True)
        var = jnp.mean(jnp.square(mm - mu), axis=1, keepdims=True)
        mm = (mm - mu) * lax.rsqrt(var + 1e-5) * lng_ref[h] + lnb_ref[h]
        out_ref[h] = _silu(mm)


def _edge_mlp(ef, gsrc, gdst, wk1, ku2, kub1, kub2,
              wv1, lu2, lub1, lub2, mlw, mlb, lng, lnb, bng, bnb, k):
    full3 = lambda shape: pl.BlockSpec(shape, lambda e: (0, 0, 0))
    e0 = k * (_EC // _BE)
    return pl.pallas_call(
        _edge_body,
        grid=(_EC // _BE,),
        in_specs=[
            pl.BlockSpec((_BE, D), lambda e: (e0 + e, 0)),
            pl.BlockSpec((_BE, TWO_D), lambda e: (e, 0)),
            pl.BlockSpec((_BE, D), lambda e: (e, 0)),
            full3((H, TWO_D, TWO_D)),
            full3((H, TWO_D, TWO_D)),
            full3((H, 1, TWO_D)),
            full3((H, 1, TWO_D)),
            full3((H, TWO_D, TWO_D)),
            full3((H, TWO_D, TWO_D)),
            full3((H, 1, TWO_D)),
            full3((H, 1, TWO_D)),
            full3((H, TWO_D, D)),
            full3((H, 1, D)),
            full3((H, 1, D)),
            full3((H, 1, D)),
            full3((H, 1, TWO_D)),
            full3((H, 1, TWO_D)),
        ],
        out_specs=pl.BlockSpec((H, _BE, D), lambda e: (0, e, 0)),
        out_shape=jax.ShapeDtypeStruct((H, _EC, D), _F32),
    )(ef, gsrc, gdst, wk1, ku2, kub1, kub2,
      wv1, lu2, lub1, lub2, mlw, mlb, lng, lnb, bng, bnb)


_SN = _EPT // _SC


def _scatter_group_body(*refs, k0, nm):
    ms = refs[:nm]
    dst_idx3, hv_hbm, ibuf, mbufs, zbuf, acc, msems = refs[nm:]
    cid = lax.axis_index("c")
    sid = lax.axis_index("s")
    base = sid * _EPT

    def zb_body(i, carry):
        for j in range(D // 16):
            zbuf[i, pl.ds(j * 16, 16)] = jnp.zeros((16,), _F32)
        return carry

    lax.fori_loop(0, _ZR, zb_body, 0)
    for j in range(_NPT // _ZR):
        pltpu.sync_copy(zbuf, acc.at[pl.ds(sid * _NPT + j * _ZR, _ZR)])
    plsc.subcore_barrier()

    for lk in range(nm):
        m_hbm = ms[lk]

        pltpu.sync_copy(dst_idx3.at[(k0 + lk) * _NS + sid], ibuf)

        def fire(j, b, m_hbm=m_hbm):
            pltpu.async_copy(m_hbm.at[cid, pl.ds(base + j * _SC, _SC)],
                             mbufs.at[b], msems.at[b])

        def drain_scatter(j, b, m_hbm=m_hbm):
            pltpu.make_async_copy(m_hbm.at[cid, pl.ds(base + j * _SC, _SC)],
                                  mbufs.at[b], msems.at[b]).wait()
            pltpu.sync_copy(mbufs.at[b], acc.at[ibuf.at[j]], add=True)

        fire(0, 0)

        def body(i, carry):
            p = lax.rem(i, 2)

            @pl.when(jnp.logical_and(i + 1 < _SN, p == 0))
            def _():
                fire(i + 1, 1)

            @pl.when(jnp.logical_and(i + 1 < _SN, p == 1))
            def _():
                fire(i + 1, 0)

            @pl.when(p == 0)
            def _():
                drain_scatter(i, 0)

            @pl.when(p == 1)
            def _():
                drain_scatter(i, 1)

            return carry

        lax.fori_loop(0, _SN, body, 0)

    plsc.subcore_barrier()
    pltpu.sync_copy(acc.at[pl.ds(sid * _NPT, _NPT)],
                    hv_hbm.at[cid, pl.ds(sid * _NPT, _NPT)])


def _scatter_group(ms, dst_idx, k0):
    mesh = plsc.VectorSubcoreMesh(core_axis_name="c", subcore_axis_name="s")
    f = functools.partial(
        pl.kernel,
        out_type=jax.ShapeDtypeStruct((H, _NPAD, D), _F32),
        mesh=mesh,
        scratch_types=[
            pltpu.VMEM((_SN, _SC), jnp.int32),
            pltpu.VMEM((2, _SC, D), _F32),
            pltpu.VMEM((_ZR, D), _F32),
            pltpu.VMEM_SHARED((_NPAD, D), _F32),
            pltpu.SemaphoreType.DMA((2,)),
        ],
    )(functools.partial(_scatter_group_body, k0=k0, nm=len(ms)))
    return f(*ms, dst_idx.reshape(_K * _NS, _SN, _SC))


_NHV = _K


def _final_body(*refs):
    x_ref = refs[0]
    hv_refs = refs[1:1 + _NHV]
    wc_ref, bc_ref, bnvg_ref, bnvb_ref, o_ref = refs[1 + _NHV:]
    hv0 = hv_refs[0][0]
    hv1 = hv_refs[0][1]
    for r in hv_refs[1:]:
        hv0 = hv0 + r[0]
        hv1 = hv1 + r[1]
    t = _dot(hv0, wc_ref[:D]) + _dot(hv1, wc_ref[D:]) + bc_ref[...]
    t = t * _INV_BN * bnvg_ref[...] + bnvb_ref[...]
    o_ref[...] = x_ref[...] + _silu(t)


def _final(x, hvs, wc, bc, bnvg, bnvb):
    return pl.pallas_call(
        _final_body,
        grid=(N // _BN,),
        in_specs=[pl.BlockSpec((_BN, D), lambda i: (i, 0))]
        + [pl.BlockSpec((H, _BN, D), lambda i: (0, i, 0))] * _NHV
        + [
            pl.BlockSpec((TWO_D, D), lambda i: (0, 0)),
            pl.BlockSpec((1, D), lambda i: (0, 0)),
            pl.BlockSpec((1, D), lambda i: (0, 0)),
            pl.BlockSpec((1, D), lambda i: (0, 0)),
        ],
        out_specs=pl.BlockSpec((_BN, D), lambda i: (i, 0)),
        out_shape=jax.ShapeDtypeStruct((N, D), _F32),
    )(x, *hvs, wc, bc, bnvg, bnvb)


def kernel(x, edge_index, edge_feature, K_v2v, V_v2v, K_e2v, V_e2v,
           ku_w1, ku_b1, ku_w2, ku_b2, lu_w1, lu_b1, lu_w2, lu_b2,
           ml_w, ml_b, ln_g, ln_b, bn_g, bn_b, wc, bc, bnv_g, bnv_b):
    eidx = edge_index.astype(jnp.int32)
    src_idx = eidx[0]
    dst_idx = eidx[1]

    wke, wve = _fold_weights(K_e2v, V_e2v, ku_w1[:, D:, :], lu_w1[:, D:, :])
    src_tab, dst_tab = _node_tables(x, K_v2v, V_v2v)

    r2 = lambda a: a.reshape(H, 1, a.shape[-1])
    b16 = lambda a: a.astype(_BF16)
    wk1 = jnp.concatenate([ku_w1[:, :D, :], wke], axis=1)
    wv1 = jnp.concatenate([lu_w1[:, :D, :], wve], axis=1)
    ew = (b16(wk1), b16(ku_w2), r2(ku_b1), r2(ku_b2),
          b16(wv1), b16(lu_w2), r2(lu_b1), r2(lu_b2),
          b16(ml_w), r2(ml_b), r2(ln_g), r2(ln_b), r2(bn_g), r2(bn_b))

    hvs = []
    for k in range(_K):
        gsrc, gdst = _gather(src_tab, dst_tab, src_idx, dst_idx, k)
        m = _edge_mlp(edge_feature, gsrc, gdst, *ew, k)
        hvs.append(_scatter_group([m], dst_idx, k))

    return _final(x, hvs, wc, bc.reshape(1, D),
                  bnv_g.reshape(1, D), bnv_b.reshape(1, D))
